# baseline (speedup 1.0000x reference)
import functools

import jax
import jax.numpy as jnp
from jax import lax
from jax.experimental import pallas as pl
from jax.experimental.pallas import tpu as pltpu
from jax.experimental.pallas import tpu_sc as plsc

_NC = 2
_NS = 16
_NW = _NC * _NS
_RING = 4


@functools.lru_cache(maxsize=None)
def _build_gather(b: int, n_neg: int, d: int):
    pos_per_w = b // _NW
    bblk = b // _NW
    ktn = d // 8
    assert bblk == 128 and d % 8 == 0 and n_neg % _RING == 0

    mesh = plsc.VectorSubcoreMesh(
        core_axis_name="c", subcore_axis_name="s",
        num_cores=_NC, num_subcores=_NS)

    @functools.partial(
        pl.kernel,
        out_type=(
            jax.ShapeDtypeStruct((b, d), jnp.float32),
            jax.ShapeDtypeStruct((n_neg, ktn, _NW, 8, 128), jnp.float32),
        ),
        mesh=mesh,
        scratch_types=[
            pltpu.VMEM((pos_per_w,), jnp.int32),
            pltpu.VMEM((pos_per_w, d), jnp.float32),
            pltpu.VMEM((n_neg, 128), jnp.int32),
            [pltpu.VMEM((128, d), jnp.float32) for _ in range(_RING)],
            [pltpu.VMEM((d, 128), jnp.float32) for _ in range(_RING)],
            [pltpu.SemaphoreType.DMA for _ in range(_RING)],
            [pltpu.SemaphoreType.DMA for _ in range(_RING)],
            pltpu.SemaphoreType.DMA,
        ],
        compiler_params=pltpu.CompilerParams(use_tc_tiling_on_sc=False,
                                             needs_layout_passes=False),
    )
    def gather_k(pos_hbm, negT_hbm, table_hbm, pos_out, negT_out,
                 pidx_v, prows_v, idsblk, rowbuf, tbuf, gsems, ssems, psem):
        wid = lax.axis_index("s") * _NC + lax.axis_index("c")

        pltpu.sync_copy(negT_hbm.at[:, pl.ds(wid * bblk, bblk)], idsblk)

        for p in range(_RING):
            pltpu.async_copy(table_hbm.at[idsblk.at[p]], rowbuf[p], gsems[p])

        pbase = wid * pos_per_w
        pltpu.sync_copy(pos_hbm.at[pl.ds(pbase, pos_per_w)], pidx_v)
        pltpu.async_copy(table_hbm.at[pidx_v], prows_v, psem).wait()
        pltpu.sync_copy(prows_v, pos_out.at[pl.ds(pbase, pos_per_w)])

        iota = lax.iota(jnp.int32, 16)
        row_idx = [iota + 16 * g for g in range(8)]
        perm = [(iota + c) & 15 for c in range(16)]

        @pl.loop(0, n_neg, step=_RING)
        def _ring(n0):
            for p in range(_RING):
                n = n0 + p
                pltpu.make_async_copy(table_hbm.at[idsblk.at[n]], rowbuf[p],
                                      gsems[p]).wait()

                @pl.when(n0 >= _RING)
                def _drain():
                    for kt in range(ktn):
                        pltpu.make_async_copy(
                            tbuf[p].at[pl.ds(kt * 8, 8)],
                            negT_out.at[n - _RING, kt, wid], ssems[p]).wait()

                @pl.loop(0, d // 16)
                def _q(q):
                    k0 = q * 16
                    cols = [perm[c] + k0 for c in range(16)]
                    for g in range(8):
                        rv = row_idx[g]
                        for c in range(16):
                            v = plsc.load_gather(rowbuf[p], [rv, cols[c]])
                            plsc.store_scatter(tbuf[p], [cols[c], rv], v)

                for kt in range(ktn):
                    pltpu.async_copy(tbuf[p].at[pl.ds(kt * 8, 8)],
                                     negT_out.at[n, kt, wid], ssems[p])

                @pl.when(n + _RING < n_neg)
                def _refill():
                    pltpu.async_copy(table_hbm.at[idsblk.at[n + _RING]],
                                     rowbuf[p], gsems[p])

        for p in range(_RING):
            n = n_neg - _RING + p
            for kt in range(ktn):
                pltpu.make_async_copy(tbuf[p].at[pl.ds(kt * 8, 8)],
                                      negT_out.at[n, kt, wid],
                                      ssems[p]).wait()

    return gather_k


def kernel(query_emb, positive_ids, negative_ids, table):
    b, n_neg = negative_ids.shape
    _, d = table.shape
    ids_t = negative_ids.T
    gather_k = _build_gather(b, n_neg, d)
    pos_emb, neg_t = gather_k(positive_ids, ids_t, table)
    neg_emb = jnp.transpose(neg_t, (2, 4, 0, 1, 3)).reshape(b, n_neg, d)
    return (query_emb, pos_emb, neg_emb)


# speedup vs baseline: 1.0296x; 1.0296x over previous
import functools

import jax
import jax.numpy as jnp
from jax import lax
from jax.experimental import pallas as pl
from jax.experimental.pallas import tpu as pltpu
from jax.experimental.pallas import tpu_sc as plsc

_NC = 2
_NS = 16
_NW = _NC * _NS
_RING = 2


@functools.lru_cache(maxsize=None)
def _build_gather(b: int, n_neg: int, d: int):
    pos_per_w = b // _NW
    bblk = b // _NW
    ktn = d // 8
    assert bblk == 128 and d % 8 == 0 and n_neg % _RING == 0

    mesh = plsc.VectorSubcoreMesh(
        core_axis_name="c", subcore_axis_name="s",
        num_cores=_NC, num_subcores=_NS)

    @functools.partial(
        pl.kernel,
        out_type=(
            jax.ShapeDtypeStruct((b, d), jnp.float32),
            jax.ShapeDtypeStruct((n_neg, ktn, _NW, 8, 128), jnp.float32),
        ),
        mesh=mesh,
        scratch_types=[
            pltpu.VMEM((pos_per_w,), jnp.int32),
            pltpu.VMEM((pos_per_w, d), jnp.float32),
            pltpu.VMEM((n_neg, 128), jnp.int32),
            [pltpu.VMEM((128, d), jnp.float32) for _ in range(_RING)],
            [pltpu.VMEM((d, 128), jnp.float32) for _ in range(_RING)],
            [pltpu.SemaphoreType.DMA for _ in range(_RING)],
            [pltpu.SemaphoreType.DMA for _ in range(_RING)],
            pltpu.SemaphoreType.DMA,
        ],
        compiler_params=pltpu.CompilerParams(use_tc_tiling_on_sc=False,
                                             needs_layout_passes=False),
    )
    def gather_k(pos_hbm, negT_hbm, table_hbm, pos_out, negT_out,
                 pidx_v, prows_v, idsblk, rowbuf, tbuf, gsems, ssems, psem):
        wid = lax.axis_index("s") * _NC + lax.axis_index("c")

        pltpu.sync_copy(negT_hbm.at[:, pl.ds(wid * bblk, bblk)], idsblk)

        for p in range(_RING):
            pltpu.async_copy(table_hbm.at[idsblk.at[p]], rowbuf[p], gsems[p])

        pbase = wid * pos_per_w
        pltpu.sync_copy(pos_hbm.at[pl.ds(pbase, pos_per_w)], pidx_v)
        pltpu.async_copy(table_hbm.at[pidx_v], prows_v, psem).wait()
        pltpu.sync_copy(prows_v, pos_out.at[pl.ds(pbase, pos_per_w)])

        iota = lax.iota(jnp.int32, 16)
        row_idx = [iota + 16 * g for g in range(8)]
        perm = [(iota + c) & 15 for c in range(16)]

        @pl.loop(0, n_neg, step=_RING)
        def _ring(n0):
            for p in range(_RING):
                n = n0 + p
                pltpu.make_async_copy(table_hbm.at[idsblk.at[n]], rowbuf[p],
                                      gsems[p]).wait()

                @pl.when(n0 >= _RING)
                def _drain():
                    for kt in range(ktn):
                        pltpu.make_async_copy(
                            tbuf[p].at[pl.ds(kt * 8, 8)],
                            negT_out.at[n - _RING, kt, wid], ssems[p]).wait()

                @pl.loop(0, d // 16)
                def _q(q):
                    k0 = q * 16
                    cols = [perm[c] + k0 for c in range(16)]
                    for g in range(8):
                        rv = row_idx[g]
                        for c in range(16):
                            v = plsc.load_gather(rowbuf[p], [rv, cols[c]])
                            plsc.store_scatter(tbuf[p], [cols[c], rv], v)

                for kt in range(ktn):
                    pltpu.async_copy(tbuf[p].at[pl.ds(kt * 8, 8)],
                                     negT_out.at[n, kt, wid], ssems[p])

                @pl.when(n + _RING < n_neg)
                def _refill():
                    pltpu.async_copy(table_hbm.at[idsblk.at[n + _RING]],
                                     rowbuf[p], gsems[p])

        for p in range(_RING):
            n = n_neg - _RING + p
            for kt in range(ktn):
                pltpu.make_async_copy(tbuf[p].at[pl.ds(kt * 8, 8)],
                                      negT_out.at[n, kt, wid],
                                      ssems[p]).wait()

    return gather_k


def kernel(query_emb, positive_ids, negative_ids, table):
    b, n_neg = negative_ids.shape
    _, d = table.shape
    ids_t = negative_ids.T
    gather_k = _build_gather(b, n_neg, d)
    pos_emb, neg_t = gather_k(positive_ids, ids_t, table)
    neg_emb = jnp.transpose(neg_t, (2, 4, 0, 1, 3)).reshape(b, n_neg, d)
    return (query_emb, pos_emb, neg_emb)


# speedup vs baseline: 1.0502x; 1.0199x over previous
import functools

import jax
import jax.numpy as jnp
from jax import lax
from jax.experimental import pallas as pl
from jax.experimental.pallas import tpu as pltpu
from jax.experimental.pallas import tpu_sc as plsc

_NC = 2
_NS = 16
_NW = _NC * _NS
_RING = 2


@functools.lru_cache(maxsize=None)
def _build_gather(b: int, n_neg: int, d: int):
    pos_per_w = b // _NW
    bblk = b // _NW
    ktn = d // 8
    assert bblk == 128 and d % 8 == 0 and n_neg % _RING == 0

    mesh = plsc.VectorSubcoreMesh(
        core_axis_name="c", subcore_axis_name="s",
        num_cores=_NC, num_subcores=_NS)

    @functools.partial(
        pl.kernel,
        out_type=(
            jax.ShapeDtypeStruct((b, d), jnp.float32),
            jax.ShapeDtypeStruct((n_neg, ktn, _NW, 8, 128), jnp.float32),
        ),
        mesh=mesh,
        scratch_types=[
            pltpu.VMEM((pos_per_w,), jnp.int32),
            pltpu.VMEM((pos_per_w, d), jnp.float32),
            pltpu.VMEM((n_neg, 128), jnp.int32),
            [pltpu.VMEM((128, d), jnp.float32) for _ in range(_RING)],
            [pltpu.VMEM((d // 8, 8, 128), jnp.float32) for _ in range(_RING)],
            [pltpu.SemaphoreType.DMA for _ in range(_RING)],
            [pltpu.SemaphoreType.DMA for _ in range(_RING)],
            pltpu.SemaphoreType.DMA,
        ],
        compiler_params=pltpu.CompilerParams(use_tc_tiling_on_sc=False,
                                             needs_layout_passes=False),
    )
    def gather_k(pos_hbm, negT_hbm, table_hbm, pos_out, negT_out,
                 pidx_v, prows_v, idsblk, rowbuf, tbuf, gsems, ssems, psem):
        wid = lax.axis_index("s") * _NC + lax.axis_index("c")

        pltpu.sync_copy(negT_hbm.at[:, pl.ds(wid * bblk, bblk)], idsblk)

        for p in range(_RING):
            pltpu.async_copy(table_hbm.at[idsblk.at[p]], rowbuf[p], gsems[p])

        pbase = wid * pos_per_w
        pltpu.sync_copy(pos_hbm.at[pl.ds(pbase, pos_per_w)], pidx_v)
        pltpu.async_copy(table_hbm.at[pidx_v], prows_v, psem).wait()
        pltpu.sync_copy(prows_v, pos_out.at[pl.ds(pbase, pos_per_w)])

        iota = lax.iota(jnp.int32, 16)
        row_idx = [iota + 16 * g for g in range(8)]
        perm = [(iota + c) & 15 for c in range(16)]

        @pl.loop(0, n_neg, step=_RING)
        def _ring(n0):
            for p in range(_RING):
                n = n0 + p
                pltpu.make_async_copy(table_hbm.at[idsblk.at[n]], rowbuf[p],
                                      gsems[p]).wait()

                @pl.when(n0 >= _RING)
                def _drain():
                    pltpu.make_async_copy(
                        tbuf[p], negT_out.at[n - _RING, :, wid],
                        ssems[p]).wait()

                @pl.loop(0, d // 16)
                def _q(q):
                    k0 = q * 16
                    cols = [perm[c] + k0 for c in range(16)]
                    kts = [cv >> 3 for cv in cols]
                    krs = [cv & 7 for cv in cols]
                    for g in range(8):
                        rv = row_idx[g]
                        for c in range(16):
                            v = plsc.load_gather(rowbuf[p], [rv, cols[c]])
                            plsc.store_scatter(tbuf[p], [kts[c], krs[c], rv],
                                               v)

                pltpu.async_copy(tbuf[p], negT_out.at[n, :, wid], ssems[p])

                @pl.when(n + _RING < n_neg)
                def _refill():
                    pltpu.async_copy(table_hbm.at[idsblk.at[n + _RING]],
                                     rowbuf[p], gsems[p])

        for p in range(_RING):
            n = n_neg - _RING + p
            pltpu.make_async_copy(tbuf[p], negT_out.at[n, :, wid],
                                  ssems[p]).wait()

    return gather_k


def kernel(query_emb, positive_ids, negative_ids, table):
    b, n_neg = negative_ids.shape
    _, d = table.shape
    ids_t = negative_ids.T
    gather_k = _build_gather(b, n_neg, d)
    pos_emb, neg_t = gather_k(positive_ids, ids_t, table)
    neg_emb = jnp.transpose(neg_t, (2, 4, 0, 1, 3)).reshape(b, n_neg, d)
    return (query_emb, pos_emb, neg_emb)
